# baseline (speedup 1.0000x reference)
import functools

import jax
import jax.numpy as jnp
from jax import lax
from jax.experimental import pallas as pl
from jax.experimental.pallas import tpu as pltpu
from jax.experimental.pallas import tpu_sc as plsc

TEMP_INV = 10.0

B = 16384
C0 = 64
C1 = 256
D = 32
NC = 2
NS = 16
NW = NC * NS
B_PER_W = B // NW
CHUNK = 128
NCHUNK = B_PER_W // CHUNK


def _sc_gather_body(uids_hbm, iids_hbm, assign0_hbm, assign1_hbm, item_hbm,
                    g0_hbm, g1_hbm, v_hbm,
                    uidx_v, iidx_v, g0_v, g1_v, v_v, sem):
    wid = lax.axis_index("s") * NC + lax.axis_index("c")
    for c in range(NCHUNK):
        base = wid * B_PER_W + c * CHUNK
        pltpu.sync_copy(uids_hbm.at[pl.ds(base, CHUNK)], uidx_v)
        pltpu.sync_copy(iids_hbm.at[pl.ds(base, CHUNK)], iidx_v)
        cp0 = pltpu.async_copy(assign0_hbm.at[uidx_v], g0_v, sem)
        cp1 = pltpu.async_copy(assign1_hbm.at[uidx_v], g1_v, sem)
        cp2 = pltpu.async_copy(item_hbm.at[iidx_v], v_v, sem)
        cp0.wait()
        cp1.wait()
        cp2.wait()
        pltpu.sync_copy(g0_v, g0_hbm.at[pl.ds(base, CHUNK)])
        pltpu.sync_copy(g1_v, g1_hbm.at[pl.ds(base, CHUNK)])
        pltpu.sync_copy(v_v, v_hbm.at[pl.ds(base, CHUNK)])


@functools.cache
def _sc_gather():
    return pl.kernel(
        _sc_gather_body,
        out_type=[
            jax.ShapeDtypeStruct((B, C0), jnp.float32),
            jax.ShapeDtypeStruct((B, C1), jnp.float32),
            jax.ShapeDtypeStruct((B, D), jnp.float32),
        ],
        mesh=plsc.VectorSubcoreMesh(core_axis_name="c", subcore_axis_name="s"),
        scratch_types=[
            pltpu.VMEM((CHUNK,), jnp.int32),
            pltpu.VMEM((CHUNK,), jnp.int32),
            pltpu.VMEM((CHUNK, C0), jnp.float32),
            pltpu.VMEM((CHUNK, C1), jnp.float32),
            pltpu.VMEM((CHUNK, D), jnp.float32),
            pltpu.SemaphoreType.DMA,
        ],
        compiler_params=pltpu.CompilerParams(use_tc_tiling_on_sc=False),
    )


def _tc_score_body(g0_ref, g1_ref, v_ref, cb0_ref, cb1_ref, out_ref):
    g0 = g0_ref[...]
    m0 = jnp.max(g0, axis=-1, keepdims=True)
    e0 = jnp.exp((g0 - m0) * TEMP_INV)
    s0 = jnp.sum(e0, axis=-1, keepdims=True)
    u0 = jnp.dot(e0, cb0_ref[...], preferred_element_type=jnp.float32,
                 precision=lax.Precision.HIGHEST) / s0
    g1 = g1_ref[...]
    m1 = jnp.max(g1, axis=-1, keepdims=True)
    e1 = jnp.exp((g1 - m1) * TEMP_INV)
    s1 = jnp.sum(e1, axis=-1, keepdims=True)
    u1 = jnp.dot(e1, cb1_ref[...], preferred_element_type=jnp.float32,
                 precision=lax.Precision.HIGHEST) / s1
    out_ref[...] = jnp.sum((u0 + u1) * v_ref[...], axis=-1, keepdims=True)


def _tc_score(g0, g1, v, cb0, cb1):
    blk = 2048
    grid = (B // blk,)
    return pl.pallas_call(
        _tc_score_body,
        grid=grid,
        in_specs=[
            pl.BlockSpec((blk, C0), lambda i: (i, 0)),
            pl.BlockSpec((blk, C1), lambda i: (i, 0)),
            pl.BlockSpec((blk, D), lambda i: (i, 0)),
            pl.BlockSpec((C0, D), lambda i: (0, 0)),
            pl.BlockSpec((C1, D), lambda i: (0, 0)),
        ],
        out_specs=pl.BlockSpec((blk, 1), lambda i: (i, 0)),
        out_shape=jax.ShapeDtypeStruct((B, 1), jnp.float32),
    )(g0, g1, v, cb0, cb1)


def kernel(X, assign0, codebook0, assign1, codebook1, item_table):
    user_ids = X[:, 0].astype(jnp.int32)
    item_ids = X[:, 1].astype(jnp.int32)
    g0, g1, v = _sc_gather()(user_ids, item_ids, assign0, assign1, item_table)
    return _tc_score(g0, g1, v, codebook0, codebook1)


# speedup vs baseline: 1.0743x; 1.0743x over previous
import functools

import jax
import jax.numpy as jnp
from jax import lax
from jax.experimental import pallas as pl
from jax.experimental.pallas import tpu as pltpu
from jax.experimental.pallas import tpu_sc as plsc

TEMP_INV = 10.0

B = 16384
C0 = 64
C1 = 256
D = 32
NC = 2
NS = 16
NW = NC * NS
B_PER_W = B // NW
CHUNK = 128
NCHUNK = B_PER_W // CHUNK


def _sc_gather_body(u_hbm, uh_hbm, iq_hbm, assign1_hbm, a0r_hbm, itr_hbm,
                    g1_hbm, g0r_hbm, vr_hbm,
                    uidx_v, uhidx_v, iqidx_v, g1_v, g0r_v, vr_v, sem):
    wid = lax.axis_index("s") * NC + lax.axis_index("c")
    for c in range(NCHUNK):
        base = wid * B_PER_W + c * CHUNK
        pltpu.sync_copy(u_hbm.at[pl.ds(base, CHUNK)], uidx_v)
        pltpu.sync_copy(uh_hbm.at[pl.ds(base, CHUNK)], uhidx_v)
        pltpu.sync_copy(iq_hbm.at[pl.ds(base, CHUNK)], iqidx_v)
        cp1 = pltpu.async_copy(assign1_hbm.at[uidx_v], g1_v, sem)
        cp0 = pltpu.async_copy(a0r_hbm.at[uhidx_v], g0r_v, sem)
        cp2 = pltpu.async_copy(itr_hbm.at[iqidx_v], vr_v, sem)
        cp1.wait()
        cp0.wait()
        cp2.wait()
        pltpu.sync_copy(g1_v, g1_hbm.at[pl.ds(base, CHUNK)])
        pltpu.sync_copy(g0r_v, g0r_hbm.at[pl.ds(base, CHUNK)])
        pltpu.sync_copy(vr_v, vr_hbm.at[pl.ds(base, CHUNK)])


@functools.cache
def _sc_gather():
    return pl.kernel(
        _sc_gather_body,
        out_type=[
            jax.ShapeDtypeStruct((B, C1), jnp.float32),
            jax.ShapeDtypeStruct((B, 128), jnp.float32),
            jax.ShapeDtypeStruct((B, 128), jnp.float32),
        ],
        mesh=plsc.VectorSubcoreMesh(core_axis_name="c", subcore_axis_name="s"),
        scratch_types=[
            pltpu.VMEM((CHUNK,), jnp.int32),
            pltpu.VMEM((CHUNK,), jnp.int32),
            pltpu.VMEM((CHUNK,), jnp.int32),
            pltpu.VMEM((CHUNK, C1), jnp.float32),
            pltpu.VMEM((CHUNK, 128), jnp.float32),
            pltpu.VMEM((CHUNK, 128), jnp.float32),
            pltpu.SemaphoreType.DMA,
        ],
        compiler_params=pltpu.CompilerParams(use_tc_tiling_on_sc=True),
    )


def _tc_score_body(g1_ref, g0r_ref, vr_ref, u_ref, i_ref, cb0d_ref, cb1_ref,
                   out_ref):
    blk = g1_ref.shape[0]
    u = u_ref[...]
    i = i_ref[...]
    lane = lax.broadcasted_iota(jnp.int32, (blk, 128), 1)

    g0 = g0r_ref[...]
    mask0 = (lane >> 6) == (u & 1)
    g0m = jnp.where(mask0, g0, -jnp.inf)
    m0 = jnp.max(g0m, axis=-1, keepdims=True)
    e0 = jnp.where(mask0, jnp.exp((g0 - m0) * TEMP_INV), 0.0)
    s0 = jnp.sum(e0, axis=-1, keepdims=True)
    u0 = jnp.dot(e0, cb0d_ref[...], preferred_element_type=jnp.float32,
                 precision=lax.Precision.HIGHEST) / s0

    g1 = g1_ref[...]
    m1 = jnp.max(g1, axis=-1, keepdims=True)
    e1 = jnp.exp((g1 - m1) * TEMP_INV)
    s1 = jnp.sum(e1, axis=-1, keepdims=True)
    u1 = jnp.dot(e1, cb1_ref[...], preferred_element_type=jnp.float32,
                 precision=lax.Precision.HIGHEST) / s1

    vr = vr_ref[...]
    maskv = (lane >> 5) == (i & 3)
    vsel = jnp.where(maskv, vr, 0.0)
    row = lax.broadcasted_iota(jnp.int32, (128, D), 0)
    col = lax.broadcasted_iota(jnp.int32, (128, D), 1)
    sel = ((row & (D - 1)) == col).astype(jnp.float32)
    v = jnp.dot(vsel, sel, preferred_element_type=jnp.float32,
                precision=lax.Precision.HIGHEST)

    out_ref[...] = jnp.sum((u0 + u1) * v, axis=-1, keepdims=True)


def _tc_score(g1, g0r, vr, u_col, i_col, cb0d, cb1):
    blk = 2048
    grid = (B // blk,)
    return pl.pallas_call(
        _tc_score_body,
        grid=grid,
        in_specs=[
            pl.BlockSpec((blk, C1), lambda i: (i, 0)),
            pl.BlockSpec((blk, 128), lambda i: (i, 0)),
            pl.BlockSpec((blk, 128), lambda i: (i, 0)),
            pl.BlockSpec((blk, 1), lambda i: (i, 0)),
            pl.BlockSpec((blk, 1), lambda i: (i, 0)),
            pl.BlockSpec((128, D), lambda i: (0, 0)),
            pl.BlockSpec((C1, D), lambda i: (0, 0)),
        ],
        out_specs=pl.BlockSpec((blk, 1), lambda i: (i, 0)),
        out_shape=jax.ShapeDtypeStruct((B, 1), jnp.float32),
    )(g1, g0r, vr, u_col, i_col, cb0d, cb1)


def kernel(X, assign0, codebook0, assign1, codebook1, item_table):
    user_ids = X[:, 0].astype(jnp.int32)
    item_ids = X[:, 1].astype(jnp.int32)
    a0r = assign0.reshape(assign0.shape[0] // 2, 128)
    itr = item_table.reshape(item_table.shape[0] // 4, 128)
    cb0d = jnp.concatenate([codebook0, codebook0], axis=0)
    g1, g0r, vr = _sc_gather()(
        user_ids, user_ids >> 1, item_ids >> 2, assign1, a0r, itr)
    return _tc_score(g1, g0r, vr, X[:, 0:1].astype(jnp.int32),
                     X[:, 1:2].astype(jnp.int32), cb0d, codebook1)


# speedup vs baseline: 1.5760x; 1.4670x over previous
import functools

import jax
import jax.numpy as jnp
from jax import lax
from jax.experimental import pallas as pl
from jax.experimental.pallas import tpu as pltpu
from jax.experimental.pallas import tpu_sc as plsc

TEMP_INV = 10.0

B = 16384
C0 = 64
C1 = 256
D = 32
NC = 2
NS = 16
NW = NC * NS
B_PER_W = B // NW
CHUNK = 128
NCHUNK = B_PER_W // CHUNK
GRP = 16


def _sc_gather_body(u_hbm, i_hbm, a1_hbm, a0_hbm, it_hbm,
                    g1_hbm, g0_hbm, v_hbm,
                    uid_v, iid_v, g1_v, g0_v, v_v,
                    sem_i, sem_g, sem_r, sem_w):
    wid = lax.axis_index("s") * NC + lax.axis_index("c")
    tbase = wid * B_PER_W
    cp_u = pltpu.async_copy(u_hbm.at[pl.ds(tbase, B_PER_W)], uid_v, sem_i)
    cp_i = pltpu.async_copy(i_hbm.at[pl.ds(tbase, B_PER_W)], iid_v, sem_i)
    cp_u.wait()
    cp_i.wait()
    for c in range(NCHUNK):
        off = c * CHUNK
        cpg = pltpu.async_copy(
            a1_hbm.at[uid_v.at[pl.ds(off, CHUNK)]], g1_v, sem_g)

        def group(g, _):
            vu = uid_v[pl.ds(off + g * GRP, GRP)]
            vi = iid_v[pl.ds(off + g * GRP, GRP)]
            cps = []
            for k in range(GRP):
                cps.append(pltpu.async_copy(
                    a0_hbm.at[pl.ds(vu[k], 1)],
                    g0_v.at[pl.ds(g * GRP + k, 1)], sem_r))
                cps.append(pltpu.async_copy(
                    it_hbm.at[pl.ds(vi[k], 1)],
                    v_v.at[pl.ds(g * GRP + k, 1)], sem_r))
            for cp in cps:
                cp.wait()
            return 0

        lax.fori_loop(0, CHUNK // GRP, group, 0)
        cpg.wait()
        wb1 = pltpu.async_copy(g1_v, g1_hbm.at[pl.ds(tbase + off, CHUNK)],
                               sem_w)
        wb0 = pltpu.async_copy(g0_v, g0_hbm.at[pl.ds(tbase + off, CHUNK)],
                               sem_w)
        wbv = pltpu.async_copy(v_v, v_hbm.at[pl.ds(tbase + off, CHUNK)],
                               sem_w)
        wb1.wait()
        wb0.wait()
        wbv.wait()


@functools.cache
def _sc_gather():
    return pl.kernel(
        _sc_gather_body,
        out_type=[
            jax.ShapeDtypeStruct((B, C1), jnp.float32),
            jax.ShapeDtypeStruct((B, C0), jnp.float32),
            jax.ShapeDtypeStruct((B, D), jnp.float32),
        ],
        mesh=plsc.VectorSubcoreMesh(core_axis_name="c", subcore_axis_name="s"),
        scratch_types=[
            pltpu.VMEM((B_PER_W,), jnp.int32),
            pltpu.VMEM((B_PER_W,), jnp.int32),
            pltpu.VMEM((CHUNK, C1), jnp.float32),
            pltpu.VMEM((CHUNK, C0), jnp.float32),
            pltpu.VMEM((CHUNK, D), jnp.float32),
            pltpu.SemaphoreType.DMA,
            pltpu.SemaphoreType.DMA,
            pltpu.SemaphoreType.DMA,
            pltpu.SemaphoreType.DMA,
        ],
        compiler_params=pltpu.CompilerParams(use_tc_tiling_on_sc=True),
    )


def _tc_score_body(g1_ref, g0_ref, v_ref, cb0_ref, cb1_ref, out_ref):
    g0 = g0_ref[...]
    m0 = jnp.max(g0, axis=-1, keepdims=True)
    e0 = jnp.exp((g0 - m0) * TEMP_INV)
    s0 = jnp.sum(e0, axis=-1, keepdims=True)
    u0 = jnp.dot(e0, cb0_ref[...], preferred_element_type=jnp.float32,
                 precision=lax.Precision.HIGHEST) / s0
    g1 = g1_ref[...]
    m1 = jnp.max(g1, axis=-1, keepdims=True)
    e1 = jnp.exp((g1 - m1) * TEMP_INV)
    s1 = jnp.sum(e1, axis=-1, keepdims=True)
    u1 = jnp.dot(e1, cb1_ref[...], preferred_element_type=jnp.float32,
                 precision=lax.Precision.HIGHEST) / s1
    out_ref[...] = jnp.sum((u0 + u1) * v_ref[...], axis=-1)


def _tc_score(g1, g0, v, cb0, cb1):
    blk = 2048
    grid = (B // blk,)
    return pl.pallas_call(
        _tc_score_body,
        grid=grid,
        in_specs=[
            pl.BlockSpec((blk, C1), lambda i: (i, 0)),
            pl.BlockSpec((blk, C0), lambda i: (i, 0)),
            pl.BlockSpec((blk, D), lambda i: (i, 0)),
            pl.BlockSpec((C0, D), lambda i: (0, 0)),
            pl.BlockSpec((C1, D), lambda i: (0, 0)),
        ],
        out_specs=pl.BlockSpec((blk,), lambda i: (i,)),
        out_shape=jax.ShapeDtypeStruct((B,), jnp.float32),
    )(g1, g0, v, cb0, cb1)


def kernel(X, assign0, codebook0, assign1, codebook1, item_table):
    user_ids = X[:, 0].astype(jnp.int32)
    item_ids = X[:, 1].astype(jnp.int32)
    g1, g0, v = _sc_gather()(
        user_ids, item_ids, assign1, assign0, item_table)
    return _tc_score(g1, g0, v, codebook0, codebook1).reshape(B, 1)
